# baseline (speedup 1.0000x reference)
import functools

import jax
import jax.numpy as jnp
from jax import lax
from jax.experimental import pallas as pl
from jax.experimental.pallas import tpu as pltpu
from jax.experimental.pallas import tpu_sc as plsc

NUM_CORES = 2
NUM_SUBCORES = 16
NW = NUM_CORES * NUM_SUBCORES
LANES = 16

VOCAB = 1000000
EMB = 64
BATCH = 4096
SEQ = 50

VH_FULL = VOCAB // 128
VH_TAIL = VOCAB - VH_FULL * 128
N_BLOCKS = VOCAB * EMB // 1024


def _mesh():
    return plsc.VectorSubcoreMesh(core_axis_name="c", subcore_axis_name="s")


@functools.lru_cache(maxsize=None)
def _make_transpose():
    @functools.partial(
        pl.kernel,
        out_type=jax.ShapeDtypeStruct((N_BLOCKS, 8, 128), jnp.float32),
        mesh=_mesh(),
        compiler_params=pltpu.CompilerParams(use_tc_tiling_on_sc=True, needs_layout_passes=False),
        scratch_types=[
            pltpu.VMEM((2, 8, 8, 128), jnp.float32),
            pltpu.VMEM((2, 8, 8, 128), jnp.float32),
            pltpu.SemaphoreType.DMA,
            pltpu.SemaphoreType.DMA,
            pltpu.SemaphoreType.DMA,
            pltpu.SemaphoreType.DMA,
        ],
    )
    def k1(tab_t, tail_t, t2, inb, outb, rs0, rs1, ws0, ws1):
        wid = lax.axis_index("s") * NUM_CORES + lax.axis_index("c")
        rsem = (rs0, rs1)
        wsem = (ws0, ws1)
        lane = lax.iota(jnp.int32, LANES)
        lane_hi = lane // 8
        lane_lo = lane % 8

        def start_reads(vh, slot, width):
            for e_hi in range(8):
                pltpu.async_copy(
                    tab_t.at[pl.ds(8 * e_hi, 8), pl.ds(vh * 128, width)],
                    inb.at[slot, e_hi].at[:, pl.ds(0, width)],
                    rsem[slot],
                )

        def wait_reads(slot, width):
            for e_hi in range(8):
                pltpu.make_async_copy(
                    tab_t.at[pl.ds(0, 8), pl.ds(0, width)],
                    inb.at[slot, e_hi].at[:, pl.ds(0, width)],
                    rsem[slot],
                ).wait()

        def transpose(slot, nq):
            src = inb.at[slot]
            for q in range(nq):
                for r in range(8):
                    for k in range(8):
                        v_lo = q * 16 + r * 2 + (1 if k >= 4 else 0)
                        i0 = lane_hi + 2 * (k % 4)
                        i2 = jnp.full((LANES,), v_lo, jnp.int32)
                        vec = plsc.load_gather(src, [i0, lane_lo, i2])
                        outb[slot, q, r, pl.ds(16 * k, LANES)] = vec

        def start_write(vh, slot, nblk):
            pltpu.async_copy(
                outb.at[slot].at[pl.ds(0, nblk)],
                t2.at[pl.ds(8 * vh, nblk)],
                wsem[slot],
            )

        def wait_write(slot, nblk):
            pltpu.make_async_copy(
                outb.at[slot].at[pl.ds(0, nblk)],
                t2.at[pl.ds(0, nblk)],
                wsem[slot],
            ).wait()

        n_i = jnp.where(wid < VH_FULL % NW, VH_FULL // NW + 1, VH_FULL // NW)

        @pl.when(n_i > 0)
        def _():
            start_reads(wid, 0, 128)

        @pl.when(n_i > 1)
        def _():
            start_reads(wid + NW, 1, 128)

        @pl.loop(0, n_i)
        def _(i):
            slot_sel = i % 2

            def body(slot):
                vh = wid + i * NW
                wait_reads(slot, 128)

                @pl.when(i >= 2)
                def _():
                    wait_write(slot, 8)

                transpose(slot, 8)
                start_write(vh, slot, 8)

                @pl.when(i + 2 < n_i)
                def _():
                    start_reads(wid + (i + 2) * NW, slot, 128)

            @pl.when(slot_sel == 0)
            def _():
                body(0)

            @pl.when(slot_sel == 1)
            def _():
                body(1)

        @pl.when(n_i > 0)
        def _():
            wait_write(0, 8)

        @pl.when(n_i > 1)
        def _():
            wait_write(1, 8)

        @pl.when(wid == NW - 1)
        def _():
            for e_hi in range(8):
                pltpu.async_copy(
                    tail_t.at[pl.ds(8 * e_hi, 8), pl.ds(0, 128)],
                    inb.at[0, e_hi],
                    rsem[0],
                )
            wait_reads(0, 128)
            transpose(0, VH_TAIL // 16)
            start_write(VH_FULL, 0, VH_TAIL // 16)
            wait_write(0, VH_TAIL // 16)

    return k1


@functools.lru_cache(maxsize=None)
def _make_gather():
    @functools.partial(
        pl.kernel,
        out_type=jax.ShapeDtypeStruct((SEQ, 8, NW, 8, 128), jnp.float32),
        mesh=_mesh(),
        compiler_params=pltpu.CompilerParams(use_tc_tiling_on_sc=False, needs_layout_passes=False),
        scratch_types=[
            pltpu.VMEM((SEQ, 128), jnp.int32),
            pltpu.VMEM((2, 128, EMB), jnp.float32),
            pltpu.VMEM((2, 8, 8, 128), jnp.float32),
            pltpu.SemaphoreType.DMA,
            pltpu.SemaphoreType.DMA,
            pltpu.SemaphoreType.DMA,
            pltpu.SemaphoreType.DMA,
        ],
    )
    def k2(t2_flat, idx_hbm, out, idx_v, rows, planes, gs0, gs1, ws0, ws1):
        wid = lax.axis_index("s") * NUM_CORES + lax.axis_index("c")
        gsem = (gs0, gs1)
        wsem = (ws0, ws1)
        lane = lax.iota(jnp.int32, LANES)

        pltpu.sync_copy(idx_hbm.at[wid], idx_v)

        def start_gather(l, slot):
            pltpu.async_copy(t2_flat.at[idx_v.at[l]], rows.at[slot], gsem[slot])

        def wait_gather(slot):
            pltpu.make_async_copy(
                t2_flat.at[idx_v.at[0]], rows.at[slot], gsem[slot]
            ).wait()

        row_base = [lane + 16 * k for k in range(8)]

        def transpose(slot):
            src = rows.at[slot]
            for e in range(EMB):
                e_col = jnp.full((LANES,), e, jnp.int32)
                for k in range(8):
                    vec = plsc.load_gather(src, [row_base[k], e_col])
                    planes[slot, e // 8, e % 8, pl.ds(16 * k, LANES)] = vec

        def start_writes(l, slot):
            for e_hi in range(8):
                pltpu.async_copy(
                    planes.at[slot, e_hi], out.at[l, e_hi, wid], wsem[slot]
                )

        def wait_writes(slot):
            for e_hi in range(8):
                pltpu.make_async_copy(
                    planes.at[slot, e_hi], out.at[0, e_hi, wid], wsem[slot]
                ).wait()

        start_gather(0, 0)
        start_gather(1, 1)

        @pl.loop(0, SEQ, step=2)
        def _(l0):
            for d in (0, 1):
                l = l0 + d
                wait_gather(d)

                @pl.when(l0 > 0)
                def _():
                    wait_writes(d)

                transpose(d)
                start_writes(l, d)

                @pl.when(l + 2 < SEQ)
                def _():
                    start_gather(l + 2, d)

        wait_writes(0)
        wait_writes(1)

    return k2


def kernel(input_, table):
    b, l = input_.shape
    vocab, emb = table.shape
    tab_t = jnp.transpose(table)
    tail_t = jnp.pad(
        jnp.transpose(lax.slice(table, (VH_FULL * 128, 0), (vocab, emb))),
        ((0, 0), (0, 128 - VH_TAIL)),
    )
    t2 = _make_transpose()(tab_t, tail_t)
    t2_flat = jnp.reshape(t2, (vocab, emb))
    idx_r = (
        input_.astype(jnp.int32).reshape(NW, b // NW, l).transpose(0, 2, 1)
    )
    o2 = _make_gather()(t2_flat, idx_r)
    return jnp.transpose(o2, (2, 4, 0, 1, 3)).reshape(b, l, emb)


# speedup vs baseline: 2.2915x; 2.2915x over previous
import functools

import jax
import jax.numpy as jnp
from jax import lax
from jax.experimental import pallas as pl
from jax.experimental.pallas import tpu as pltpu
from jax.experimental.pallas import tpu_sc as plsc

NUM_CORES = 2
NUM_SUBCORES = 16
NW = NUM_CORES * NUM_SUBCORES
LANES = 16

VOCAB = 1000000
EMB = 64
BATCH = 4096
SEQ = 50

VH_FULL = VOCAB // 128
VH_TAIL = VOCAB - VH_FULL * 128
N_BLOCKS = VOCAB * EMB // 1024


def _mesh():
    return plsc.VectorSubcoreMesh(core_axis_name="c", subcore_axis_name="s")


@functools.lru_cache(maxsize=None)
def _make_transpose():
    @functools.partial(
        pl.kernel,
        out_type=jax.ShapeDtypeStruct((N_BLOCKS, 8, 128), jnp.float32),
        mesh=_mesh(),
        compiler_params=pltpu.CompilerParams(use_tc_tiling_on_sc=True, needs_layout_passes=False),
        scratch_types=[
            pltpu.VMEM((2, 8, 8, 128), jnp.float32),
            pltpu.VMEM((2, 8, 8, 128), jnp.float32),
            pltpu.SemaphoreType.DMA,
            pltpu.SemaphoreType.DMA,
            pltpu.SemaphoreType.DMA,
            pltpu.SemaphoreType.DMA,
        ],
    )
    def k1(tab_t, tail_t, t2, inb, outb, rs0, rs1, ws0, ws1):
        wid = lax.axis_index("s") * NUM_CORES + lax.axis_index("c")
        rsem = (rs0, rs1)
        wsem = (ws0, ws1)
        lane = lax.iota(jnp.int32, LANES)
        lane_hi = lane // 8
        lane_lo = lane % 8

        def start_reads(vh, slot, width):
            for e_hi in range(8):
                pltpu.async_copy(
                    tab_t.at[pl.ds(8 * e_hi, 8), pl.ds(vh * 128, width)],
                    inb.at[slot, e_hi].at[:, pl.ds(0, width)],
                    rsem[slot],
                )

        def wait_reads(slot, width):
            for e_hi in range(8):
                pltpu.make_async_copy(
                    tab_t.at[pl.ds(0, 8), pl.ds(0, width)],
                    inb.at[slot, e_hi].at[:, pl.ds(0, width)],
                    rsem[slot],
                ).wait()

        e_base = [lane_hi + 2 * m for m in range(4)]

        def transpose(slot, nq):
            src = inb.at[slot]

            @plsc.parallel_loop(0, nq * 8, unroll=2)
            def _(t):
                q = t // 8
                r = t % 8
                s = q * 16 + r * 2
                i2a = jnp.full((LANES,), 0, jnp.int32) + s
                i2b = i2a + 1
                for k in range(8):
                    i2 = i2b if k >= 4 else i2a
                    vec = plsc.load_gather(src, [e_base[k % 4], lane_lo, i2])
                    outb[slot, q, r, pl.ds(16 * k, LANES)] = vec

        def start_write(vh, slot, nblk):
            pltpu.async_copy(
                outb.at[slot].at[pl.ds(0, nblk)],
                t2.at[pl.ds(8 * vh, nblk)],
                wsem[slot],
            )

        def wait_write(slot, nblk):
            pltpu.make_async_copy(
                outb.at[slot].at[pl.ds(0, nblk)],
                t2.at[pl.ds(0, nblk)],
                wsem[slot],
            ).wait()

        n_i = jnp.where(wid < VH_FULL % NW, VH_FULL // NW + 1, VH_FULL // NW)

        @pl.when(n_i > 0)
        def _():
            start_reads(wid, 0, 128)

        @pl.when(n_i > 1)
        def _():
            start_reads(wid + NW, 1, 128)

        @pl.loop(0, n_i)
        def _(i):
            slot_sel = i % 2

            def body(slot):
                vh = wid + i * NW
                wait_reads(slot, 128)

                @pl.when(i >= 2)
                def _():
                    wait_write(slot, 8)

                transpose(slot, 8)
                start_write(vh, slot, 8)

                @pl.when(i + 2 < n_i)
                def _():
                    start_reads(wid + (i + 2) * NW, slot, 128)

            @pl.when(slot_sel == 0)
            def _():
                body(0)

            @pl.when(slot_sel == 1)
            def _():
                body(1)

        @pl.when(n_i > 0)
        def _():
            wait_write(0, 8)

        @pl.when(n_i > 1)
        def _():
            wait_write(1, 8)

        @pl.when(wid == NW - 1)
        def _():
            for e_hi in range(8):
                pltpu.async_copy(
                    tail_t.at[pl.ds(8 * e_hi, 8), pl.ds(0, 128)],
                    inb.at[0, e_hi],
                    rsem[0],
                )
            wait_reads(0, 128)
            transpose(0, VH_TAIL // 16)
            start_write(VH_FULL, 0, VH_TAIL // 16)
            wait_write(0, VH_TAIL // 16)

    return k1


@functools.lru_cache(maxsize=None)
def _make_gather():
    @functools.partial(
        pl.kernel,
        out_type=jax.ShapeDtypeStruct((SEQ, 8, NW, 8, 128), jnp.float32),
        mesh=_mesh(),
        compiler_params=pltpu.CompilerParams(use_tc_tiling_on_sc=False, needs_layout_passes=False),
        scratch_types=[
            pltpu.VMEM((SEQ, 128), jnp.int32),
            pltpu.VMEM((2, 128, EMB), jnp.float32),
            pltpu.VMEM((2, 8, 8, 128), jnp.float32),
            pltpu.SemaphoreType.DMA,
            pltpu.SemaphoreType.DMA,
            pltpu.SemaphoreType.DMA,
            pltpu.SemaphoreType.DMA,
        ],
    )
    def k2(t2_flat, idx_hbm, out, idx_v, rows, planes, gs0, gs1, ws0, ws1):
        wid = lax.axis_index("s") * NUM_CORES + lax.axis_index("c")
        gsem = (gs0, gs1)
        wsem = (ws0, ws1)
        lane = lax.iota(jnp.int32, LANES)

        pltpu.sync_copy(idx_hbm.at[wid], idx_v)

        def start_gather(l, slot):
            pltpu.async_copy(t2_flat.at[idx_v.at[l]], rows.at[slot], gsem[slot])

        def wait_gather(slot):
            pltpu.make_async_copy(
                t2_flat.at[idx_v.at[0]], rows.at[slot], gsem[slot]
            ).wait()

        row_base = [lane + 16 * k for k in range(8)]

        def transpose(slot):
            src = rows.at[slot]

            @plsc.parallel_loop(0, EMB, unroll=2)
            def _(e):
                e_col = jnp.full((LANES,), 0, jnp.int32) + e
                e_hi = e // 8
                e_lo = e % 8
                for k in range(8):
                    vec = plsc.load_gather(src, [row_base[k], e_col])
                    planes[slot, e_hi, e_lo, pl.ds(16 * k, LANES)] = vec

        def start_writes(l, slot):
            for e_hi in range(8):
                pltpu.async_copy(
                    planes.at[slot, e_hi], out.at[l, e_hi, wid], wsem[slot]
                )

        def wait_writes(slot):
            for e_hi in range(8):
                pltpu.make_async_copy(
                    planes.at[slot, e_hi], out.at[0, e_hi, wid], wsem[slot]
                ).wait()

        start_gather(0, 0)
        start_gather(1, 1)

        @pl.loop(0, SEQ, step=2)
        def _(l0):
            for d in (0, 1):
                l = l0 + d
                wait_gather(d)

                @pl.when(l0 > 0)
                def _():
                    wait_writes(d)

                transpose(d)
                start_writes(l, d)

                @pl.when(l + 2 < SEQ)
                def _():
                    start_gather(l + 2, d)

        wait_writes(0)
        wait_writes(1)

    return k2


def kernel(input_, table):
    b, l = input_.shape
    vocab, emb = table.shape
    tab_t = jnp.transpose(table)
    tail_t = jnp.pad(
        jnp.transpose(lax.slice(table, (VH_FULL * 128, 0), (vocab, emb))),
        ((0, 0), (0, 128 - VH_TAIL)),
    )
    t2 = _make_transpose()(tab_t, tail_t)
    t2_flat = jnp.reshape(t2, (vocab, emb))
    idx_r = (
        input_.astype(jnp.int32).reshape(NW, b // NW, l).transpose(0, 2, 1)
    )
    o2 = _make_gather()(t2_flat, idx_r)
    return jnp.transpose(o2, (2, 4, 0, 1, 3)).reshape(b, l, emb)


# speedup vs baseline: 2.5554x; 1.1152x over previous
import functools

import jax
import jax.numpy as jnp
from jax import lax
from jax.experimental import pallas as pl
from jax.experimental.pallas import tpu as pltpu
from jax.experimental.pallas import tpu_sc as plsc

NUM_CORES = 2
NUM_SUBCORES = 16
NW = NUM_CORES * NUM_SUBCORES
LANES = 16

VOCAB = 1000000
EMB = 64
BATCH = 4096
SEQ = 50

VH_FULL = VOCAB // 128
VH_TAIL = VOCAB - VH_FULL * 128
N_BLOCKS = VOCAB * EMB // 1024


def _mesh():
    return plsc.VectorSubcoreMesh(core_axis_name="c", subcore_axis_name="s")


@functools.lru_cache(maxsize=None)
def _make_transpose():
    @functools.partial(
        pl.kernel,
        out_type=jax.ShapeDtypeStruct((N_BLOCKS, 8, 128), jnp.float32),
        mesh=_mesh(),
        compiler_params=pltpu.CompilerParams(use_tc_tiling_on_sc=True, needs_layout_passes=False),
        scratch_types=[
            pltpu.VMEM((2, 8, 8, 129), jnp.float32),
            pltpu.VMEM((2, 8, 8, 128), jnp.float32),
            pltpu.SemaphoreType.DMA,
            pltpu.SemaphoreType.DMA,
            pltpu.SemaphoreType.DMA,
            pltpu.SemaphoreType.DMA,
        ],
    )
    def k1(tab_t, tail_t, t2, inb, outb, rs0, rs1, ws0, ws1):
        wid = lax.axis_index("s") * NUM_CORES + lax.axis_index("c")
        rsem = (rs0, rs1)
        wsem = (ws0, ws1)
        lane = lax.iota(jnp.int32, LANES)
        lane_hi = lane // 8
        lane_lo = lane % 8

        def start_reads(vh, slot, width):
            for e_hi in range(8):
                pltpu.async_copy(
                    tab_t.at[pl.ds(8 * e_hi, 8), pl.ds(vh * 128, width)],
                    inb.at[slot, e_hi].at[:, pl.ds(0, width)],
                    rsem[slot],
                )

        def wait_reads(slot, width):
            for e_hi in range(8):
                pltpu.make_async_copy(
                    tab_t.at[pl.ds(0, 8), pl.ds(0, width)],
                    inb.at[slot, e_hi].at[:, pl.ds(0, width)],
                    rsem[slot],
                ).wait()

        e_base = [lane_hi + 2 * m for m in range(4)]

        def transpose(slot, nq):
            src = inb.at[slot]

            @plsc.parallel_loop(0, nq * 8, unroll=4)
            def _(t):
                q = t // 8
                r = t % 8
                s = q * 16 + r * 2
                i2a = jnp.full((LANES,), 0, jnp.int32) + s
                i2b = i2a + 1
                for k in range(8):
                    i2 = i2b if k >= 4 else i2a
                    vec = plsc.load_gather(src, [e_base[k % 4], lane_lo, i2])
                    outb[slot, q, r, pl.ds(16 * k, LANES)] = vec

        def start_write(vh, slot, nblk):
            pltpu.async_copy(
                outb.at[slot].at[pl.ds(0, nblk)],
                t2.at[pl.ds(8 * vh, nblk)],
                wsem[slot],
            )

        def wait_write(slot, nblk):
            pltpu.make_async_copy(
                outb.at[slot].at[pl.ds(0, nblk)],
                t2.at[pl.ds(0, nblk)],
                wsem[slot],
            ).wait()

        n_i = jnp.where(wid < VH_FULL % NW, VH_FULL // NW + 1, VH_FULL // NW)

        @pl.when(n_i > 0)
        def _():
            start_reads(wid, 0, 128)

        @pl.when(n_i > 1)
        def _():
            start_reads(wid + NW, 1, 128)

        @pl.loop(0, n_i)
        def _(i):
            slot_sel = i % 2

            def body(slot):
                vh = wid + i * NW
                wait_reads(slot, 128)

                @pl.when(i >= 2)
                def _():
                    wait_write(slot, 8)

                transpose(slot, 8)
                start_write(vh, slot, 8)

                @pl.when(i + 2 < n_i)
                def _():
                    start_reads(wid + (i + 2) * NW, slot, 128)

            @pl.when(slot_sel == 0)
            def _():
                body(0)

            @pl.when(slot_sel == 1)
            def _():
                body(1)

        @pl.when(n_i > 0)
        def _():
            wait_write(0, 8)

        @pl.when(n_i > 1)
        def _():
            wait_write(1, 8)

        @pl.when(wid == NW - 1)
        def _():
            for e_hi in range(8):
                pltpu.async_copy(
                    tail_t.at[pl.ds(8 * e_hi, 8), pl.ds(0, 128)],
                    inb.at[0, e_hi].at[:, pl.ds(0, 128)],
                    rsem[0],
                )
            wait_reads(0, 128)
            transpose(0, VH_TAIL // 16)
            start_write(VH_FULL, 0, VH_TAIL // 16)
            wait_write(0, VH_TAIL // 16)

    return k1


@functools.lru_cache(maxsize=None)
def _make_gather():
    @functools.partial(
        pl.kernel,
        out_type=jax.ShapeDtypeStruct((SEQ, 8, NW, 8, 128), jnp.float32),
        mesh=_mesh(),
        compiler_params=pltpu.CompilerParams(use_tc_tiling_on_sc=False, needs_layout_passes=False),
        scratch_types=[
            pltpu.VMEM((SEQ, 128), jnp.int32),
            pltpu.VMEM((2, 128, EMB), jnp.float32),
            pltpu.VMEM((2, 8, 8, 129), jnp.float32),
            pltpu.SemaphoreType.DMA,
            pltpu.SemaphoreType.DMA,
            pltpu.SemaphoreType.DMA,
            pltpu.SemaphoreType.DMA,
        ],
    )
    def k2(t2_flat, idx_hbm, out, idx_v, rows, planes, gs0, gs1, ws0, ws1):
        wid = lax.axis_index("s") * NUM_CORES + lax.axis_index("c")
        gsem = (gs0, gs1)
        wsem = (ws0, ws1)
        lane = lax.iota(jnp.int32, LANES)

        pltpu.sync_copy(idx_hbm.at[wid], idx_v)

        def start_gather(l, slot):
            pltpu.async_copy(t2_flat.at[idx_v.at[l]], rows.at[slot], gsem[slot])

        def wait_gather(slot):
            pltpu.make_async_copy(
                t2_flat.at[idx_v.at[0]], rows.at[slot], gsem[slot]
            ).wait()

        e_hi_idx = [lane // 8 + 2 * j for j in range(4)]
        e_lo_idx = lane % 8

        def transpose(slot):
            src = rows.at[slot]
            dst = planes.at[slot]

            @plsc.parallel_loop(0, 128, unroll=4)
            def _(b):
                b_vec = jnp.full((LANES,), 0, jnp.int32) + b
                for j in range(4):
                    vec = src[b, pl.ds(16 * j, LANES)]
                    plsc.store_scatter(dst, [e_hi_idx[j], e_lo_idx, b_vec], vec)

        def start_writes(l, slot):
            for e_hi in range(8):
                pltpu.async_copy(
                    planes.at[slot, e_hi].at[:, pl.ds(0, 128)],
                    out.at[l, e_hi, wid], wsem[slot]
                )

        def wait_writes(slot):
            for e_hi in range(8):
                pltpu.make_async_copy(
                    planes.at[slot, e_hi].at[:, pl.ds(0, 128)],
                    out.at[0, e_hi, wid], wsem[slot]
                ).wait()

        start_gather(0, 0)
        start_gather(1, 1)

        @pl.loop(0, SEQ, step=2)
        def _(l0):
            for d in (0, 1):
                l = l0 + d
                wait_gather(d)

                @pl.when(l0 > 0)
                def _():
                    wait_writes(d)

                transpose(d)
                start_writes(l, d)

                @pl.when(l + 2 < SEQ)
                def _():
                    start_gather(l + 2, d)

        wait_writes(0)
        wait_writes(1)

    return k2


def kernel(input_, table):
    b, l = input_.shape
    vocab, emb = table.shape
    tab_t = jnp.transpose(table)
    tail_t = jnp.pad(
        jnp.transpose(lax.slice(table, (VH_FULL * 128, 0), (vocab, emb))),
        ((0, 0), (0, 128 - VH_TAIL)),
    )
    t2 = _make_transpose()(tab_t, tail_t)
    t2_flat = jnp.reshape(t2, (vocab, emb))
    idx_r = (
        input_.astype(jnp.int32).reshape(NW, b // NW, l).transpose(0, 2, 1)
    )
    o2 = _make_gather()(t2_flat, idx_r)
    return jnp.transpose(o2, (2, 4, 0, 1, 3)).reshape(b, l, emb)


# speedup vs baseline: 2.6033x; 1.0188x over previous
import functools

import jax
import jax.numpy as jnp
from jax import lax
from jax.experimental import pallas as pl
from jax.experimental.pallas import tpu as pltpu
from jax.experimental.pallas import tpu_sc as plsc

NUM_CORES = 2
NUM_SUBCORES = 16
NW = NUM_CORES * NUM_SUBCORES
LANES = 16

VOCAB = 1000000
EMB = 64
BATCH = 4096
SEQ = 50

VH_FULL = VOCAB // 128
VH_TAIL = VOCAB - VH_FULL * 128
N_BLOCKS = VOCAB * EMB // 1024


def _mesh():
    return plsc.VectorSubcoreMesh(core_axis_name="c", subcore_axis_name="s")


@functools.lru_cache(maxsize=None)
def _make_transpose():
    @functools.partial(
        pl.kernel,
        out_type=jax.ShapeDtypeStruct((N_BLOCKS, 8, 128), jnp.float32),
        mesh=_mesh(),
        compiler_params=pltpu.CompilerParams(use_tc_tiling_on_sc=True, needs_layout_passes=False),
        scratch_types=[
            pltpu.VMEM((2, 8, 8, 129), jnp.float32),
            pltpu.VMEM((2, 8, 8, 128), jnp.float32),
            pltpu.SemaphoreType.DMA,
            pltpu.SemaphoreType.DMA,
            pltpu.SemaphoreType.DMA,
            pltpu.SemaphoreType.DMA,
        ],
    )
    def k1(tab_t, tail_t, t2, inb, outb, rs0, rs1, ws0, ws1):
        wid = lax.axis_index("s") * NUM_CORES + lax.axis_index("c")
        rsem = (rs0, rs1)
        wsem = (ws0, ws1)
        lane = lax.iota(jnp.int32, LANES)
        lane_hi = lane // 8
        lane_lo = lane % 8

        def start_reads(vh, slot, width):
            for e_hi in range(8):
                pltpu.async_copy(
                    tab_t.at[pl.ds(8 * e_hi, 8), pl.ds(vh * 128, width)],
                    inb.at[slot, e_hi].at[:, pl.ds(0, width)],
                    rsem[slot],
                )

        def wait_reads(slot, width):
            for e_hi in range(8):
                pltpu.make_async_copy(
                    tab_t.at[pl.ds(0, 8), pl.ds(0, width)],
                    inb.at[slot, e_hi].at[:, pl.ds(0, width)],
                    rsem[slot],
                ).wait()

        e_base = [lane_hi + 2 * m for m in range(4)]

        def transpose(slot, nq):
            src = inb.at[slot]

            @plsc.parallel_loop(0, nq * 8, unroll=8)
            def _(t):
                q = t // 8
                r = t % 8
                s = q * 16 + r * 2
                i2a = jnp.full((LANES,), 0, jnp.int32) + s
                i2b = i2a + 1
                for k in range(8):
                    i2 = i2b if k >= 4 else i2a
                    vec = plsc.load_gather(src, [e_base[k % 4], lane_lo, i2])
                    outb[slot, q, r, pl.ds(16 * k, LANES)] = vec

        def start_write(vh, slot, nblk):
            pltpu.async_copy(
                outb.at[slot].at[pl.ds(0, nblk)],
                t2.at[pl.ds(8 * vh, nblk)],
                wsem[slot],
            )

        def wait_write(slot, nblk):
            pltpu.make_async_copy(
                outb.at[slot].at[pl.ds(0, nblk)],
                t2.at[pl.ds(0, nblk)],
                wsem[slot],
            ).wait()

        n_i = jnp.where(wid < VH_FULL % NW, VH_FULL // NW + 1, VH_FULL // NW)

        @pl.when(n_i > 0)
        def _():
            start_reads(wid, 0, 128)

        @pl.when(n_i > 1)
        def _():
            start_reads(wid + NW, 1, 128)

        @pl.loop(0, n_i)
        def _(i):
            slot_sel = i % 2

            def body(slot):
                vh = wid + i * NW
                wait_reads(slot, 128)

                @pl.when(i >= 2)
                def _():
                    wait_write(slot, 8)

                transpose(slot, 8)
                start_write(vh, slot, 8)

                @pl.when(i + 2 < n_i)
                def _():
                    start_reads(wid + (i + 2) * NW, slot, 128)

            @pl.when(slot_sel == 0)
            def _():
                body(0)

            @pl.when(slot_sel == 1)
            def _():
                body(1)

        @pl.when(n_i > 0)
        def _():
            wait_write(0, 8)

        @pl.when(n_i > 1)
        def _():
            wait_write(1, 8)

        @pl.when(wid == NW - 1)
        def _():
            for e_hi in range(8):
                pltpu.async_copy(
                    tail_t.at[pl.ds(8 * e_hi, 8), pl.ds(0, 128)],
                    inb.at[0, e_hi].at[:, pl.ds(0, 128)],
                    rsem[0],
                )
            wait_reads(0, 128)
            transpose(0, VH_TAIL // 16)
            start_write(VH_FULL, 0, VH_TAIL // 16)
            wait_write(0, VH_TAIL // 16)

    return k1


@functools.lru_cache(maxsize=None)
def _make_gather():
    @functools.partial(
        pl.kernel,
        out_type=jax.ShapeDtypeStruct((SEQ, 8, NW, 8, 128), jnp.float32),
        mesh=_mesh(),
        compiler_params=pltpu.CompilerParams(use_tc_tiling_on_sc=False, needs_layout_passes=False),
        scratch_types=[
            pltpu.VMEM((SEQ, 128), jnp.int32),
            pltpu.VMEM((2, 128, EMB), jnp.float32),
            pltpu.VMEM((2, 8, 8, 129), jnp.float32),
            pltpu.SemaphoreType.DMA,
            pltpu.SemaphoreType.DMA,
            pltpu.SemaphoreType.DMA,
            pltpu.SemaphoreType.DMA,
        ],
    )
    def k2(t2_flat, idx_hbm, out, idx_v, rows, planes, gs0, gs1, ws0, ws1):
        wid = lax.axis_index("s") * NUM_CORES + lax.axis_index("c")
        gsem = (gs0, gs1)
        wsem = (ws0, ws1)
        lane = lax.iota(jnp.int32, LANES)

        pltpu.sync_copy(idx_hbm.at[wid], idx_v)

        def start_gather(l, slot):
            pltpu.async_copy(t2_flat.at[idx_v.at[l]], rows.at[slot], gsem[slot])

        def wait_gather(slot):
            pltpu.make_async_copy(
                t2_flat.at[idx_v.at[0]], rows.at[slot], gsem[slot]
            ).wait()

        e_hi_idx = [lane // 8 + 2 * j for j in range(4)]
        e_lo_idx = lane % 8

        def transpose(slot):
            src = rows.at[slot]
            dst = planes.at[slot]

            @plsc.parallel_loop(0, 128, unroll=8)
            def _(b):
                b_vec = jnp.full((LANES,), 0, jnp.int32) + b
                for j in range(4):
                    vec = src[b, pl.ds(16 * j, LANES)]
                    plsc.store_scatter(dst, [e_hi_idx[j], e_lo_idx, b_vec], vec)

        def start_writes(l, slot):
            for e_hi in range(8):
                pltpu.async_copy(
                    planes.at[slot, e_hi].at[:, pl.ds(0, 128)],
                    out.at[l, e_hi, wid], wsem[slot]
                )

        def wait_writes(slot):
            for e_hi in range(8):
                pltpu.make_async_copy(
                    planes.at[slot, e_hi].at[:, pl.ds(0, 128)],
                    out.at[0, e_hi, wid], wsem[slot]
                ).wait()

        start_gather(0, 0)
        start_gather(1, 1)

        @pl.loop(0, SEQ, step=2)
        def _(l0):
            for d in (0, 1):
                l = l0 + d
                wait_gather(d)

                @pl.when(l0 > 0)
                def _():
                    wait_writes(d)

                transpose(d)
                start_writes(l, d)

                @pl.when(l + 2 < SEQ)
                def _():
                    start_gather(l + 2, d)

        wait_writes(0)
        wait_writes(1)

    return k2


def kernel(input_, table):
    b, l = input_.shape
    vocab, emb = table.shape
    tab_t = jnp.transpose(table)
    tail_t = jnp.pad(
        jnp.transpose(lax.slice(table, (VH_FULL * 128, 0), (vocab, emb))),
        ((0, 0), (0, 128 - VH_TAIL)),
    )
    t2 = _make_transpose()(tab_t, tail_t)
    t2_flat = jnp.reshape(t2, (vocab, emb))
    idx_r = (
        input_.astype(jnp.int32).reshape(NW, b // NW, l).transpose(0, 2, 1)
    )
    o2 = _make_gather()(t2_flat, idx_r)
    return jnp.transpose(o2, (2, 4, 0, 1, 3)).reshape(b, l, emb)


# speedup vs baseline: 2.7790x; 1.0675x over previous
import functools

import jax
import jax.numpy as jnp
from jax import lax
from jax.experimental import pallas as pl
from jax.experimental.pallas import tpu as pltpu
from jax.experimental.pallas import tpu_sc as plsc

NUM_CORES = 2
NUM_SUBCORES = 16
NW = NUM_CORES * NUM_SUBCORES
CHUNK = 128


@functools.lru_cache(maxsize=None)
def _make_gather(n_rows, emb, n_chunks):
    b_per_w = n_rows // NW
    mesh = plsc.VectorSubcoreMesh(core_axis_name="c", subcore_axis_name="s")

    @functools.partial(
        pl.kernel,
        out_type=jax.ShapeDtypeStruct((n_rows, emb), jnp.float32),
        mesh=mesh,
        compiler_params=pltpu.CompilerParams(use_tc_tiling_on_sc=False),
        scratch_types=[
            pltpu.VMEM((n_chunks, CHUNK), jnp.int32),
            pltpu.VMEM((CHUNK, emb), jnp.float32),
            pltpu.SemaphoreType.DMA,
        ],
    )
    def k(idx_hbm, table_hbm, out_hbm, idx_v, rows_v, sem):
        wid = lax.axis_index("s") * NUM_CORES + lax.axis_index("c")
        pltpu.sync_copy(idx_hbm.at[wid], idx_v)
        base = wid * b_per_w

        @pl.loop(0, n_chunks)
        def _(j):
            pltpu.async_copy(table_hbm.at[idx_v.at[j]], rows_v, sem).wait()
            pltpu.sync_copy(rows_v, out_hbm.at[pl.ds(base + j * CHUNK, CHUNK)])

    return k


def kernel(input_, table):
    b, l = input_.shape
    vocab, emb = table.shape
    n = b * l
    idx = input_.reshape(-1).astype(jnp.int32)
    pad = (-n) % (NW * CHUNK)
    if pad:
        fill = (jnp.arange(pad, dtype=jnp.int32) * 61) % vocab
        idx = jnp.concatenate([idx, fill])
    total = n + pad
    n_chunks = total // (NW * CHUNK)
    idx3 = idx.reshape(NW, n_chunks, CHUNK)
    out = _make_gather(total, emb, n_chunks)(idx3, table)
    return out[:n].reshape(b, l, emb)
